# baseline (speedup 1.0000x reference)
import functools

import jax
import jax.numpy as jnp
from jax import lax
from jax.experimental import pallas as pl
from jax.experimental.pallas import tpu as pltpu
from jax.experimental.pallas import tpu_sc as plsc

TILE = 256
FC = 2048
GT = 512
CH = 48
NW = 32


def _gating_body(x_ref, wgt_ref, bias_ref, out_ref):
    logits = jnp.dot(x_ref[...], wgt_ref[...],
                     preferred_element_type=jnp.float32)
    logits = logits + bias_ref[...]
    e = logits.shape[1]
    m = jnp.max(logits, axis=1, keepdims=True)
    ii = lax.broadcasted_iota(jnp.int32, logits.shape, 1)
    cand = jnp.where(logits >= m, ii, e)
    out_ref[0, 0, :] = jnp.min(cand, axis=1)


def _gating(x_flat, wg_t, bias2d):
    t, d = x_flat.shape
    grid = t // GT
    out = pl.pallas_call(
        _gating_body,
        grid=(grid,),
        in_specs=[
            pl.BlockSpec((GT, d), lambda i: (i, 0)),
            pl.BlockSpec(wg_t.shape, lambda i: (0, 0)),
            pl.BlockSpec(bias2d.shape, lambda i: (0, 0)),
        ],
        out_specs=pl.BlockSpec((1, 1, GT), lambda i: (i, 0, 0)),
        out_shape=jax.ShapeDtypeStruct((grid, 1, GT), jnp.int32),
    )(x_flat, wg_t, bias2d)
    return out.reshape(t)


def _ffn_body(te_ref, act_ref, xs_ref, w1_ref, b1_ref, w2_ref, b2_ref,
              out_ref):
    t = pl.program_id(0)
    f = pl.program_id(1)
    e = te_ref[t]

    @pl.when(act_ref[t] == 1)
    def _():
        xt = xs_ref[...].astype(jnp.bfloat16)
        w1 = w1_ref[0]
        h = lax.dot_general(xt, w1, (((1,), (1,)), ((), ())),
                            preferred_element_type=jnp.float32)
        h = jax.nn.relu(h + b1_ref[e, pl.ds(f * FC, FC)][None, :])
        w2 = w2_ref[0]
        part = lax.dot_general(h.astype(jnp.bfloat16), w2,
                               (((1,), (1,)), ((), ())),
                               preferred_element_type=jnp.float32)

        @pl.when(f == 0)
        def _():
            out_ref[...] = part + b2_ref[e][None, :]

        @pl.when(f != 0)
        def _():
            out_ref[...] += part


def _ffn(te, act, xs, w1, b1, w2, b2, nt):
    tp, d = xs.shape
    e_num, f_dim, _ = w1.shape
    nf = f_dim // FC
    grid_spec = pltpu.PrefetchScalarGridSpec(
        num_scalar_prefetch=2,
        grid=(nt, nf),
        in_specs=[
            pl.BlockSpec((TILE, d), lambda t, f, te, act: (t, 0)),
            pl.BlockSpec((1, FC, d),
                         lambda t, f, te, act: (te[t], f * act[t], 0)),
            pl.BlockSpec((e_num, f_dim), lambda t, f, te, act: (0, 0)),
            pl.BlockSpec((1, d, FC),
                         lambda t, f, te, act: (te[t], 0, f * act[t])),
            pl.BlockSpec((e_num, d), lambda t, f, te, act: (0, 0)),
        ],
        out_specs=pl.BlockSpec((TILE, d), lambda t, f, te, act: (t, 0)),
    )
    return pl.pallas_call(
        _ffn_body,
        grid_spec=grid_spec,
        out_shape=jax.ShapeDtypeStruct((tp, d), jnp.float32),
        compiler_params=pltpu.CompilerParams(
            dimension_semantics=("arbitrary", "arbitrary")),
    )(te, act, xs, w1, b1, w2, b2)


def _make_gather(t, d, tp):
    npw = tp // (NW * CH)
    mesh = plsc.VectorSubcoreMesh(core_axis_name="c", subcore_axis_name="s")

    @functools.partial(
        pl.kernel, mesh=mesh,
        out_type=jax.ShapeDtypeStruct((tp, d), jnp.float32),
        scratch_types=[
            pltpu.VMEM((npw, CH), jnp.int32),
            pltpu.VMEM((CH, d), jnp.float32),
            pltpu.VMEM((CH, d), jnp.float32),
            pltpu.SemaphoreType.DMA,
            pltpu.SemaphoreType.DMA,
            pltpu.SemaphoreType.DMA,
            pltpu.SemaphoreType.DMA,
        ],
    )
    def gather(x_hbm, gidx_hbm, xs_hbm, idx_v, b0, b1, g0, g1, o0, o1):
        wid = lax.axis_index("s") * 2 + lax.axis_index("c")
        bufs = (b0, b1)
        gsems = (g0, g1)
        osems = (o0, o1)
        pltpu.sync_copy(gidx_hbm.at[wid], idx_v)
        pltpu.async_copy(x_hbm.at[idx_v.at[0]], bufs[0], gsems[0])
        if npw > 1:
            pltpu.async_copy(x_hbm.at[idx_v.at[1]], bufs[1], gsems[1])
        for c in range(npw):
            s = c % 2
            r = wid * npw + c
            pltpu.make_async_copy(x_hbm.at[idx_v.at[s]], bufs[s],
                                  gsems[s]).wait()
            pltpu.async_copy(bufs[s], xs_hbm.at[pl.ds(r * CH, CH)], osems[s])
            if c + 2 < npw:
                pltpu.make_async_copy(
                    bufs[s], xs_hbm.at[pl.ds(r * CH, CH)], osems[s]).wait()
                pltpu.async_copy(x_hbm.at[idx_v.at[c + 2]], bufs[s], gsems[s])
        for c in range(max(npw - 2, 0), npw):
            s = c % 2
            r = wid * npw + c
            pltpu.make_async_copy(
                bufs[s], xs_hbm.at[pl.ds(r * CH, CH)], osems[s]).wait()

    return gather


def _make_scatter(t, d, tp):
    npw = tp // (NW * CH)
    mesh = plsc.VectorSubcoreMesh(core_axis_name="c", subcore_axis_name="s")

    @functools.partial(
        pl.kernel, mesh=mesh,
        out_type=jax.ShapeDtypeStruct((t + 8, d), jnp.float32),
        scratch_types=[
            pltpu.VMEM((npw, CH), jnp.int32),
            pltpu.VMEM((CH, d), jnp.float32),
            pltpu.VMEM((CH, d), jnp.float32),
            pltpu.SemaphoreType.DMA,
            pltpu.SemaphoreType.DMA,
            pltpu.SemaphoreType.DMA,
            pltpu.SemaphoreType.DMA,
        ],
    )
    def scatter(ys_hbm, sidx_hbm, out_hbm, idx_v, b0, b1, i0, i1, s0, s1):
        wid = lax.axis_index("s") * 2 + lax.axis_index("c")
        bufs = (b0, b1)
        isems = (i0, i1)
        ssems = (s0, s1)
        pltpu.sync_copy(sidx_hbm.at[wid], idx_v)
        r0 = wid * npw
        pltpu.async_copy(ys_hbm.at[pl.ds(r0 * CH, CH)], bufs[0], isems[0])
        if npw > 1:
            pltpu.async_copy(ys_hbm.at[pl.ds((r0 + 1) * CH, CH)], bufs[1],
                             isems[1])
        for c in range(npw):
            s = c % 2
            r = wid * npw + c
            pltpu.make_async_copy(
                ys_hbm.at[pl.ds(r * CH, CH)], bufs[s], isems[s]).wait()
            pltpu.async_copy(bufs[s], out_hbm.at[idx_v.at[c]], ssems[s])
            if c + 2 < npw:
                pltpu.make_async_copy(
                    bufs[s], out_hbm.at[idx_v.at[c]], ssems[s]).wait()
                pltpu.async_copy(ys_hbm.at[pl.ds((r + 2) * CH, CH)], bufs[s],
                                 isems[s])
        for c in range(max(npw - 2, 0), npw):
            s = c % 2
            pltpu.make_async_copy(
                bufs[s], out_hbm.at[idx_v.at[c]], ssems[s]).wait()

    return scatter


def kernel(x, Wg, bg, W1, b1, W2, b2, expert_bias):
    b, s, d = x.shape
    e_num, f_dim, _ = W1.shape
    t = b * s
    nt_data = t // TILE
    nt = nt_data + e_num
    tp = nt * TILE

    x_flat = x.reshape(t, d)

    wg_t = Wg.T
    bias2d = (bg + expert_bias).reshape(1, e_num)
    top = _gating(x_flat, wg_t, bias2d)

    i32 = jnp.int32
    perm = jnp.argsort(top).astype(i32)
    counts = jnp.bincount(top, length=e_num).astype(i32)
    offs = jnp.concatenate(
        [jnp.zeros((1,), i32), jnp.cumsum(counts)[:-1].astype(i32)])
    ptiles = (counts + TILE - 1) // TILE
    cumt = jnp.cumsum(ptiles).astype(i32)
    total_tiles = cumt[-1]
    tidx = jnp.arange(nt, dtype=i32)
    te = jnp.searchsorted(cumt, tidx, side="right").astype(i32)
    act = (tidx < total_tiles).astype(i32)
    e_last = jnp.take(te, total_tiles - 1)
    te = jnp.where(act == 1, te, e_last).astype(i32)
    poff = jnp.concatenate(
        [jnp.zeros((1,), i32),
         jnp.cumsum(ptiles * TILE)[:-1].astype(i32)])

    j = jnp.arange(tp, dtype=i32)
    tj = j // TILE
    ej = jnp.take(te, tj)
    lp = j - jnp.take(poff, ej)
    real = ((lp >= 0) & (lp < jnp.take(counts, ej))
            & (jnp.take(act, tj) == 1))
    src = jnp.take(perm, jnp.clip(jnp.take(offs, ej) + lp, 0, t - 1))
    gidx = jnp.where(real, src, 0).astype(i32)
    sidx = jnp.where(real, src, t).astype(i32)

    npw = tp // (NW * CH)
    xs = _make_gather(t, d, tp)(x_flat, gidx.reshape(NW, npw, CH))

    w1_bf = W1.astype(jnp.bfloat16)
    w2_bf = W2.astype(jnp.bfloat16)
    ys = _ffn(te, act, xs, w1_bf, b1, w2_bf, b2, nt)

    out_pad = _make_scatter(t, d, tp)(ys, sidx.reshape(NW, npw, CH))
    return out_pad[:t].reshape(b, s, d)


# speedup vs baseline: 1.4091x; 1.4091x over previous
import functools

import jax
import jax.numpy as jnp
from jax import lax
from jax.experimental import pallas as pl
from jax.experimental.pallas import tpu as pltpu
from jax.experimental.pallas import tpu_sc as plsc

TILE = 256
FC = 2048
GT = 512
CH = 48
NW = 32


def _gating_body(x_ref, wgt_ref, bias_ref, out_ref):
    logits = jnp.dot(x_ref[...], wgt_ref[...],
                     preferred_element_type=jnp.float32)
    logits = logits + bias_ref[...]
    e = logits.shape[1]
    m = jnp.max(logits, axis=1, keepdims=True)
    ii = lax.broadcasted_iota(jnp.int32, logits.shape, 1)
    cand = jnp.where(logits >= m, ii, e)
    out_ref[0, 0, :] = jnp.min(cand, axis=1)


def _gating(x_flat, wg_t, bias2d):
    t, d = x_flat.shape
    grid = t // GT
    out = pl.pallas_call(
        _gating_body,
        grid=(grid,),
        in_specs=[
            pl.BlockSpec((GT, d), lambda i: (i, 0)),
            pl.BlockSpec(wg_t.shape, lambda i: (0, 0)),
            pl.BlockSpec(bias2d.shape, lambda i: (0, 0)),
        ],
        out_specs=pl.BlockSpec((1, 1, GT), lambda i: (i, 0, 0)),
        out_shape=jax.ShapeDtypeStruct((grid, 1, GT), jnp.int32),
    )(x_flat, wg_t, bias2d)
    return out.reshape(t)


def _ffn_body(te_ref, act_ref, xs_ref, w1_ref, b1_ref, w2_ref, b2_ref,
              out_ref):
    t = pl.program_id(0)
    f = pl.program_id(1)
    e = te_ref[t]

    @pl.when(act_ref[t] == 1)
    def _():
        xt = xs_ref[...].astype(jnp.bfloat16)
        w1 = w1_ref[0]
        h = lax.dot_general(xt, w1, (((1,), (1,)), ((), ())),
                            preferred_element_type=jnp.float32)
        h = jax.nn.relu(h + b1_ref[e, pl.ds(f * FC, FC)][None, :])
        w2 = w2_ref[0]
        part = lax.dot_general(h.astype(jnp.bfloat16), w2,
                               (((1,), (1,)), ((), ())),
                               preferred_element_type=jnp.float32)

        @pl.when(f == 0)
        def _():
            out_ref[...] = part + b2_ref[e][None, :]

        @pl.when(f != 0)
        def _():
            out_ref[...] += part


def _ffn(te, act, xs, w1, b1, w2, b2, nt):
    tp, d = xs.shape
    e_num, f_dim, _ = w1.shape
    nf = f_dim // FC
    grid_spec = pltpu.PrefetchScalarGridSpec(
        num_scalar_prefetch=2,
        grid=(nt, nf),
        in_specs=[
            pl.BlockSpec((TILE, d), lambda t, f, te, act: (t, 0)),
            pl.BlockSpec((1, FC, d),
                         lambda t, f, te, act: (te[t], f * act[t], 0)),
            pl.BlockSpec((e_num, f_dim), lambda t, f, te, act: (0, 0)),
            pl.BlockSpec((1, d, FC),
                         lambda t, f, te, act: (te[t], 0, f * act[t])),
            pl.BlockSpec((e_num, d), lambda t, f, te, act: (0, 0)),
        ],
        out_specs=pl.BlockSpec((TILE, d), lambda t, f, te, act: (t, 0)),
    )
    return pl.pallas_call(
        _ffn_body,
        grid_spec=grid_spec,
        out_shape=jax.ShapeDtypeStruct((tp, d), jnp.float32),
        compiler_params=pltpu.CompilerParams(
            dimension_semantics=("arbitrary", "arbitrary")),
    )(te, act, xs, w1, b1, w2, b2)


def _make_gather(t, d, tp):
    npw = tp // (NW * CH)
    mesh = plsc.VectorSubcoreMesh(core_axis_name="c", subcore_axis_name="s")

    @functools.partial(
        pl.kernel, mesh=mesh,
        out_type=jax.ShapeDtypeStruct((tp, d), jnp.float32),
        scratch_types=[
            pltpu.VMEM((npw, CH), jnp.int32),
            pltpu.VMEM((CH, d), jnp.float32),
            pltpu.VMEM((CH, d), jnp.float32),
            pltpu.SemaphoreType.DMA,
            pltpu.SemaphoreType.DMA,
            pltpu.SemaphoreType.DMA,
            pltpu.SemaphoreType.DMA,
        ],
    )
    def gather(x_hbm, gidx_hbm, xs_hbm, idx_v, b0, b1, g0, g1, o0, o1):
        wid = lax.axis_index("s") * 2 + lax.axis_index("c")
        bufs = (b0, b1)
        gsems = (g0, g1)
        osems = (o0, o1)
        pltpu.sync_copy(gidx_hbm.at[wid], idx_v)
        pltpu.async_copy(x_hbm.at[idx_v.at[0]], bufs[0], gsems[0])
        if npw > 1:
            pltpu.async_copy(x_hbm.at[idx_v.at[1]], bufs[1], gsems[1])
        for c in range(npw):
            s = c % 2
            r = wid * npw + c
            pltpu.make_async_copy(x_hbm.at[idx_v.at[s]], bufs[s],
                                  gsems[s]).wait()
            pltpu.async_copy(bufs[s], xs_hbm.at[pl.ds(r * CH, CH)], osems[s])
            if c + 2 < npw:
                pltpu.make_async_copy(
                    bufs[s], xs_hbm.at[pl.ds(r * CH, CH)], osems[s]).wait()
                pltpu.async_copy(x_hbm.at[idx_v.at[c + 2]], bufs[s], gsems[s])
        for c in range(max(npw - 2, 0), npw):
            s = c % 2
            r = wid * npw + c
            pltpu.make_async_copy(
                bufs[s], xs_hbm.at[pl.ds(r * CH, CH)], osems[s]).wait()

    return gather


def _make_scatter(t, d, tp):
    npw = tp // (NW * CH)
    mesh = plsc.VectorSubcoreMesh(core_axis_name="c", subcore_axis_name="s")

    @functools.partial(
        pl.kernel, mesh=mesh,
        out_type=jax.ShapeDtypeStruct((tp, d), jnp.float32),
        scratch_types=[
            pltpu.VMEM((npw, CH), jnp.int32),
            pltpu.VMEM((CH, d), jnp.float32),
            pltpu.VMEM((CH, d), jnp.float32),
            pltpu.SemaphoreType.DMA,
            pltpu.SemaphoreType.DMA,
            pltpu.SemaphoreType.DMA,
            pltpu.SemaphoreType.DMA,
        ],
    )
    def scatter(ys_hbm, sidx_hbm, out_hbm, idx_v, b0, b1, i0, i1, s0, s1):
        wid = lax.axis_index("s") * 2 + lax.axis_index("c")
        bufs = (b0, b1)
        isems = (i0, i1)
        ssems = (s0, s1)
        pltpu.sync_copy(sidx_hbm.at[wid], idx_v)
        r0 = wid * npw
        pltpu.async_copy(ys_hbm.at[pl.ds(r0 * CH, CH)], bufs[0], isems[0])
        if npw > 1:
            pltpu.async_copy(ys_hbm.at[pl.ds((r0 + 1) * CH, CH)], bufs[1],
                             isems[1])
        for c in range(npw):
            s = c % 2
            r = wid * npw + c
            pltpu.make_async_copy(
                ys_hbm.at[pl.ds(r * CH, CH)], bufs[s], isems[s]).wait()
            pltpu.async_copy(bufs[s], out_hbm.at[idx_v.at[c]], ssems[s])
            if c + 2 < npw:
                pltpu.make_async_copy(
                    bufs[s], out_hbm.at[idx_v.at[c]], ssems[s]).wait()
                pltpu.async_copy(ys_hbm.at[pl.ds((r + 2) * CH, CH)], bufs[s],
                                 isems[s])
        for c in range(max(npw - 2, 0), npw):
            s = c % 2
            pltpu.make_async_copy(
                bufs[s], out_hbm.at[idx_v.at[c]], ssems[s]).wait()

    return scatter


def kernel(x, Wg, bg, W1, b1, W2, b2, expert_bias):
    b, s, d = x.shape
    e_num, f_dim, _ = W1.shape
    t = b * s
    nt_data = t // TILE
    nt = nt_data + e_num
    tp = nt * TILE

    x_flat = x.reshape(t, d)

    wg_t = Wg.T
    bias2d = (bg + expert_bias).reshape(1, e_num)
    top = _gating(x_flat, wg_t, bias2d)

    i32 = jnp.int32
    perm = jnp.argsort(top).astype(i32)
    counts = jnp.bincount(top, length=e_num).astype(i32)
    offs = jnp.concatenate(
        [jnp.zeros((1,), i32), jnp.cumsum(counts)[:-1].astype(i32)])
    ptiles = (counts + TILE - 1) // TILE
    cumt = jnp.cumsum(ptiles).astype(i32)
    total_tiles = cumt[-1]
    tidx = jnp.arange(nt, dtype=i32)
    te = jnp.searchsorted(cumt, tidx, side="right").astype(i32)
    act = (tidx < total_tiles).astype(i32)
    e_last = jnp.take(te, total_tiles - 1)
    te = jnp.where(act == 1, te, e_last).astype(i32)
    poff = jnp.concatenate(
        [jnp.zeros((1,), i32),
         jnp.cumsum(ptiles * TILE)[:-1].astype(i32)])

    j = jnp.arange(tp, dtype=i32)
    tj = j // TILE
    ej = jnp.take(te, tj)
    lp = j - jnp.take(poff, ej)
    real = ((lp >= 0) & (lp < jnp.take(counts, ej))
            & (jnp.take(act, tj) == 1))
    src = jnp.take(perm, jnp.clip(jnp.take(offs, ej) + lp, 0, t - 1))
    gidx = jnp.where(real, src, j % t).astype(i32)
    sidx = jnp.where(real, src, t + (j % (tp - t))).astype(i32)

    npw = tp // (NW * CH)
    xs = _make_gather(t, d, tp)(x_flat, gidx.reshape(NW, npw, CH))

    w1_bf = W1.astype(jnp.bfloat16)
    w2_bf = W2.astype(jnp.bfloat16)
    ys = _ffn(te, act, xs, w1_bf, b1, w2_bf, b2, nt)

    out_pad = _make_scatter(t, d, tp)(ys, sidx.reshape(NW, npw, CH))
    return out_pad[:t].reshape(b, s, d)


# speedup vs baseline: 1.5508x; 1.1005x over previous
import functools

import jax
import jax.numpy as jnp
from jax import lax
from jax.experimental import pallas as pl
from jax.experimental.pallas import tpu as pltpu
from jax.experimental.pallas import tpu_sc as plsc

TILE = 256
FC = 1024
GT = 512
CH = 48
NW = 32


def _gating_body(x_ref, wgt_ref, bias_ref, out_ref):
    logits = jnp.dot(x_ref[...], wgt_ref[...],
                     preferred_element_type=jnp.float32)
    logits = logits + bias_ref[...]
    e = logits.shape[1]
    m = jnp.max(logits, axis=1, keepdims=True)
    ii = lax.broadcasted_iota(jnp.int32, logits.shape, 1)
    cand = jnp.where(logits >= m, ii, e)
    out_ref[0, 0, :] = jnp.min(cand, axis=1)


def _gating(x_flat, wg_t, bias2d):
    t, d = x_flat.shape
    grid = t // GT
    out = pl.pallas_call(
        _gating_body,
        grid=(grid,),
        in_specs=[
            pl.BlockSpec((GT, d), lambda i: (i, 0)),
            pl.BlockSpec(wg_t.shape, lambda i: (0, 0)),
            pl.BlockSpec(bias2d.shape, lambda i: (0, 0)),
        ],
        out_specs=pl.BlockSpec((1, 1, GT), lambda i: (i, 0, 0)),
        out_shape=jax.ShapeDtypeStruct((grid, 1, GT), jnp.int32),
    )(x_flat, wg_t, bias2d)
    return out.reshape(t)


def _ffn_body(te_ref, act_ref, xs_ref, w1_ref, b1_ref, w2_ref, b2_ref,
              out_ref, acc_ref):
    f = pl.program_id(0)
    t = pl.program_id(1)
    nf = pl.num_programs(0)
    e = te_ref[t]

    @pl.when(act_ref[t] == 1)
    def _():
        xt = xs_ref[pl.ds(t * TILE, TILE), :].astype(jnp.float32)
        w1 = w1_ref[0]
        h = lax.dot_general(xt, w1, (((1,), (1,)), ((), ())),
                            preferred_element_type=jnp.float32)
        h = jax.nn.relu(h + b1_ref[e, pl.ds(f * FC, FC)][None, :])
        w2 = w2_ref[0]
        part = lax.dot_general(h, w2, (((1,), (1,)), ((), ())),
                               preferred_element_type=jnp.float32)

        @pl.when(f == 0)
        def _():
            acc_ref[pl.ds(t * TILE, TILE), :] = part

        @pl.when((f != 0) & (f != nf - 1))
        def _():
            acc_ref[pl.ds(t * TILE, TILE), :] += part

        @pl.when(f == nf - 1)
        def _():
            out_ref[...] = (acc_ref[pl.ds(t * TILE, TILE), :] + part
                            + b2_ref[e][None, :])


def _ffn(te, act, xs, w1, b1, w2, b2, nt):
    tp, d = xs.shape
    e_num, f_dim, _ = w1.shape
    nf = f_dim // FC
    grid_spec = pltpu.PrefetchScalarGridSpec(
        num_scalar_prefetch=2,
        grid=(nf, nt),
        in_specs=[
            pl.BlockSpec((tp, d), lambda f, t, te, act: (0, 0)),
            pl.BlockSpec((1, FC, d), lambda f, t, te, act: (te[t], f, 0)),
            pl.BlockSpec((e_num, f_dim), lambda f, t, te, act: (0, 0)),
            pl.BlockSpec((1, d, FC), lambda f, t, te, act: (te[t], 0, f)),
            pl.BlockSpec((e_num, d), lambda f, t, te, act: (0, 0)),
        ],
        out_specs=pl.BlockSpec(
            (TILE, d),
            lambda f, t, te, act: (jnp.where(f == nf - 1, t, 0), 0)),
        scratch_shapes=[pltpu.VMEM((tp, d), jnp.float32)],
    )
    return pl.pallas_call(
        _ffn_body,
        grid_spec=grid_spec,
        out_shape=jax.ShapeDtypeStruct((tp, d), jnp.float32),
        compiler_params=pltpu.CompilerParams(
            dimension_semantics=("arbitrary", "arbitrary")),
    )(te, act, xs, w1, b1, w2, b2)


def _make_gather(t, d, tp, dtype):
    npw = tp // (NW * CH)
    mesh = plsc.VectorSubcoreMesh(core_axis_name="c", subcore_axis_name="s")

    @functools.partial(
        pl.kernel, mesh=mesh,
        out_type=jax.ShapeDtypeStruct((tp, d), dtype),
        scratch_types=[
            pltpu.VMEM((npw, CH), jnp.int32),
            pltpu.VMEM((CH, d), dtype),
            pltpu.VMEM((CH, d), dtype),
            pltpu.SemaphoreType.DMA,
            pltpu.SemaphoreType.DMA,
            pltpu.SemaphoreType.DMA,
            pltpu.SemaphoreType.DMA,
        ],
    )
    def gather(x_hbm, gidx_hbm, xs_hbm, idx_v, b0, b1, g0, g1, o0, o1):
        wid = lax.axis_index("s") * 2 + lax.axis_index("c")
        bufs = (b0, b1)
        gsems = (g0, g1)
        osems = (o0, o1)
        pltpu.sync_copy(gidx_hbm.at[wid], idx_v)
        pltpu.async_copy(x_hbm.at[idx_v.at[0]], bufs[0], gsems[0])
        if npw > 1:
            pltpu.async_copy(x_hbm.at[idx_v.at[1]], bufs[1], gsems[1])
        for c in range(npw):
            s = c % 2
            r = wid * npw + c
            pltpu.make_async_copy(x_hbm.at[idx_v.at[s]], bufs[s],
                                  gsems[s]).wait()
            pltpu.async_copy(bufs[s], xs_hbm.at[pl.ds(r * CH, CH)], osems[s])
            if c + 2 < npw:
                pltpu.make_async_copy(
                    bufs[s], xs_hbm.at[pl.ds(r * CH, CH)], osems[s]).wait()
                pltpu.async_copy(x_hbm.at[idx_v.at[c + 2]], bufs[s], gsems[s])
        for c in range(max(npw - 2, 0), npw):
            s = c % 2
            r = wid * npw + c
            pltpu.make_async_copy(
                bufs[s], xs_hbm.at[pl.ds(r * CH, CH)], osems[s]).wait()

    return gather


def _make_scatter(t, d, tp):
    npw = tp // (NW * CH)
    mesh = plsc.VectorSubcoreMesh(core_axis_name="c", subcore_axis_name="s")

    @functools.partial(
        pl.kernel, mesh=mesh,
        out_type=jax.ShapeDtypeStruct((tp, d), jnp.float32),
        scratch_types=[
            pltpu.VMEM((npw, CH), jnp.int32),
            pltpu.VMEM((CH, d), jnp.float32),
            pltpu.VMEM((CH, d), jnp.float32),
            pltpu.SemaphoreType.DMA,
            pltpu.SemaphoreType.DMA,
            pltpu.SemaphoreType.DMA,
            pltpu.SemaphoreType.DMA,
        ],
    )
    def scatter(ys_hbm, sidx_hbm, out_hbm, idx_v, b0, b1, i0, i1, s0, s1):
        wid = lax.axis_index("s") * 2 + lax.axis_index("c")
        bufs = (b0, b1)
        isems = (i0, i1)
        ssems = (s0, s1)
        pltpu.sync_copy(sidx_hbm.at[wid], idx_v)
        r0 = wid * npw
        pltpu.async_copy(ys_hbm.at[pl.ds(r0 * CH, CH)], bufs[0], isems[0])
        if npw > 1:
            pltpu.async_copy(ys_hbm.at[pl.ds((r0 + 1) * CH, CH)], bufs[1],
                             isems[1])
        for c in range(npw):
            s = c % 2
            r = wid * npw + c
            pltpu.make_async_copy(
                ys_hbm.at[pl.ds(r * CH, CH)], bufs[s], isems[s]).wait()
            pltpu.async_copy(bufs[s], out_hbm.at[idx_v.at[c]], ssems[s])
            if c + 2 < npw:
                pltpu.make_async_copy(
                    bufs[s], out_hbm.at[idx_v.at[c]], ssems[s]).wait()
                pltpu.async_copy(ys_hbm.at[pl.ds((r + 2) * CH, CH)], bufs[s],
                                 isems[s])
        for c in range(max(npw - 2, 0), npw):
            s = c % 2
            pltpu.make_async_copy(
                bufs[s], out_hbm.at[idx_v.at[c]], ssems[s]).wait()

    return scatter


def kernel(x, Wg, bg, W1, b1, W2, b2, expert_bias):
    b, s, d = x.shape
    e_num, f_dim, _ = W1.shape
    t = b * s
    nt_data = t // TILE
    nt = nt_data + e_num
    tp = nt * TILE

    x_flat = x.reshape(t, d)

    wg_t = Wg.T
    bias2d = (bg + expert_bias).reshape(1, e_num)
    top = _gating(x_flat, wg_t, bias2d)

    i32 = jnp.int32
    perm = jnp.argsort(top).astype(i32)
    counts = jnp.bincount(top, length=e_num).astype(i32)
    offs = jnp.concatenate(
        [jnp.zeros((1,), i32), jnp.cumsum(counts)[:-1].astype(i32)])
    ptiles = (counts + TILE - 1) // TILE
    cumt = jnp.cumsum(ptiles).astype(i32)
    total_tiles = cumt[-1]
    tidx = jnp.arange(nt, dtype=i32)
    te = jnp.searchsorted(cumt, tidx, side="right").astype(i32)
    act = (tidx < total_tiles).astype(i32)
    e_last = jnp.take(te, total_tiles - 1)
    te = jnp.where(act == 1, te, e_last).astype(i32)
    poff = jnp.concatenate(
        [jnp.zeros((1,), i32),
         jnp.cumsum(ptiles * TILE)[:-1].astype(i32)])

    j = jnp.arange(tp, dtype=i32)
    tj = j // TILE
    ej = jnp.take(te, tj)
    lp = j - jnp.take(poff, ej)
    real = ((lp >= 0) & (lp < jnp.take(counts, ej))
            & (jnp.take(act, tj) == 1))
    src = jnp.take(perm, jnp.clip(jnp.take(offs, ej) + lp, 0, t - 1))
    gidx = jnp.where(real, src, j % t).astype(i32)
    sidx = jnp.where(real, src, t + (j % (tp - t))).astype(i32)

    npw = tp // (NW * CH)
    xs = _make_gather(t, d, tp, jnp.float32)(
        x_flat, gidx.reshape(NW, npw, CH)).astype(jnp.bfloat16)

    ys = _ffn(te, act, xs, W1, b1, W2, b2, nt)

    out_pad = _make_scatter(t, d, tp)(ys, sidx.reshape(NW, npw, CH))
    return out_pad[:t].reshape(b, s, d)


# speedup vs baseline: 1.5518x; 1.0007x over previous
import functools

import jax
import jax.numpy as jnp
from jax import lax
from jax.experimental import pallas as pl
from jax.experimental.pallas import tpu as pltpu
from jax.experimental.pallas import tpu_sc as plsc

TILE = 256
FC = 1024
GT = 512
CH = 48
NW = 32


def _gating_body(x_ref, wgt_ref, bias_ref, out_ref):
    logits = jnp.dot(x_ref[...], wgt_ref[...],
                     preferred_element_type=jnp.float32)
    logits = logits + bias_ref[...]
    e = logits.shape[1]
    m = jnp.max(logits, axis=1, keepdims=True)
    ii = lax.broadcasted_iota(jnp.int32, logits.shape, 1)
    cand = jnp.where(logits >= m, ii, e)
    out_ref[0, 0, :] = jnp.min(cand, axis=1)


def _gating(x_flat, wg_t, bias2d):
    t, d = x_flat.shape
    grid = t // GT
    out = pl.pallas_call(
        _gating_body,
        grid=(grid,),
        in_specs=[
            pl.BlockSpec((GT, d), lambda i: (i, 0)),
            pl.BlockSpec(wg_t.shape, lambda i: (0, 0)),
            pl.BlockSpec(bias2d.shape, lambda i: (0, 0)),
        ],
        out_specs=pl.BlockSpec((1, 1, GT), lambda i: (i, 0, 0)),
        out_shape=jax.ShapeDtypeStruct((grid, 1, GT), jnp.int32),
    )(x_flat, wg_t, bias2d)
    return out.reshape(t)


def _ffn_body(te_ref, act_ref, xs_ref, w1_ref, b1_ref, w2_ref, b2_ref,
              out_ref, acc_ref):
    f = pl.program_id(0)
    t = pl.program_id(1)
    nf = pl.num_programs(0)
    e = te_ref[t]

    @pl.when(act_ref[t] == 1)
    def _():
        xt = xs_ref[pl.ds(t * TILE, TILE), :]
        w1 = w1_ref[0].astype(jnp.bfloat16)
        h = lax.dot_general(xt, w1, (((1,), (1,)), ((), ())),
                            preferred_element_type=jnp.float32)
        h = jax.nn.relu(h + b1_ref[e, pl.ds(f * FC, FC)][None, :])
        w2 = w2_ref[0].astype(jnp.bfloat16)
        part = lax.dot_general(h.astype(jnp.bfloat16), w2,
                               (((1,), (1,)), ((), ())),
                               preferred_element_type=jnp.float32)

        @pl.when(f == 0)
        def _():
            acc_ref[pl.ds(t * TILE, TILE), :] = part

        @pl.when((f != 0) & (f != nf - 1))
        def _():
            acc_ref[pl.ds(t * TILE, TILE), :] += part

        @pl.when(f == nf - 1)
        def _():
            out_ref[...] = (acc_ref[pl.ds(t * TILE, TILE), :] + part
                            + b2_ref[e][None, :])


def _ffn(te, act, xs, w1, b1, w2, b2, nt):
    tp, d = xs.shape
    e_num, f_dim, _ = w1.shape
    nf = f_dim // FC
    grid_spec = pltpu.PrefetchScalarGridSpec(
        num_scalar_prefetch=2,
        grid=(nf, nt),
        in_specs=[
            pl.BlockSpec((tp, d), lambda f, t, te, act: (0, 0)),
            pl.BlockSpec((1, FC, d), lambda f, t, te, act: (te[t], f, 0)),
            pl.BlockSpec((e_num, f_dim), lambda f, t, te, act: (0, 0)),
            pl.BlockSpec((1, d, FC), lambda f, t, te, act: (te[t], 0, f)),
            pl.BlockSpec((e_num, d), lambda f, t, te, act: (0, 0)),
        ],
        out_specs=pl.BlockSpec(
            (TILE, d),
            lambda f, t, te, act: (jnp.where(f == nf - 1, t, 0), 0)),
        scratch_shapes=[pltpu.VMEM((tp, d), jnp.float32)],
    )
    return pl.pallas_call(
        _ffn_body,
        grid_spec=grid_spec,
        out_shape=jax.ShapeDtypeStruct((tp, d), jnp.float32),
        compiler_params=pltpu.CompilerParams(
            dimension_semantics=("arbitrary", "arbitrary")),
    )(te, act, xs, w1, b1, w2, b2)


def _make_gather(t, d, tp, dtype):
    npw = tp // (NW * CH)
    mesh = plsc.VectorSubcoreMesh(core_axis_name="c", subcore_axis_name="s")

    @functools.partial(
        pl.kernel, mesh=mesh,
        out_type=jax.ShapeDtypeStruct((tp, d), dtype),
        scratch_types=[
            pltpu.VMEM((npw, CH), jnp.int32),
            pltpu.VMEM((CH, d), dtype),
            pltpu.VMEM((CH, d), dtype),
            pltpu.SemaphoreType.DMA,
            pltpu.SemaphoreType.DMA,
            pltpu.SemaphoreType.DMA,
            pltpu.SemaphoreType.DMA,
        ],
    )
    def gather(x_hbm, gidx_hbm, xs_hbm, idx_v, b0, b1, g0, g1, o0, o1):
        wid = lax.axis_index("s") * 2 + lax.axis_index("c")
        bufs = (b0, b1)
        gsems = (g0, g1)
        osems = (o0, o1)
        pltpu.sync_copy(gidx_hbm.at[wid], idx_v)
        pltpu.async_copy(x_hbm.at[idx_v.at[0]], bufs[0], gsems[0])
        if npw > 1:
            pltpu.async_copy(x_hbm.at[idx_v.at[1]], bufs[1], gsems[1])
        for c in range(npw):
            s = c % 2
            r = wid * npw + c
            pltpu.make_async_copy(x_hbm.at[idx_v.at[s]], bufs[s],
                                  gsems[s]).wait()
            pltpu.async_copy(bufs[s], xs_hbm.at[pl.ds(r * CH, CH)], osems[s])
            if c + 2 < npw:
                pltpu.make_async_copy(
                    bufs[s], xs_hbm.at[pl.ds(r * CH, CH)], osems[s]).wait()
                pltpu.async_copy(x_hbm.at[idx_v.at[c + 2]], bufs[s], gsems[s])
        for c in range(max(npw - 2, 0), npw):
            s = c % 2
            r = wid * npw + c
            pltpu.make_async_copy(
                bufs[s], xs_hbm.at[pl.ds(r * CH, CH)], osems[s]).wait()

    return gather


def _make_scatter(t, d, tp):
    npw = tp // (NW * CH)
    mesh = plsc.VectorSubcoreMesh(core_axis_name="c", subcore_axis_name="s")

    @functools.partial(
        pl.kernel, mesh=mesh,
        out_type=jax.ShapeDtypeStruct((tp, d), jnp.float32),
        scratch_types=[
            pltpu.VMEM((npw, CH), jnp.int32),
            pltpu.VMEM((CH, d), jnp.float32),
            pltpu.VMEM((CH, d), jnp.float32),
            pltpu.SemaphoreType.DMA,
            pltpu.SemaphoreType.DMA,
            pltpu.SemaphoreType.DMA,
            pltpu.SemaphoreType.DMA,
        ],
    )
    def scatter(ys_hbm, sidx_hbm, out_hbm, idx_v, b0, b1, i0, i1, s0, s1):
        wid = lax.axis_index("s") * 2 + lax.axis_index("c")
        bufs = (b0, b1)
        isems = (i0, i1)
        ssems = (s0, s1)
        pltpu.sync_copy(sidx_hbm.at[wid], idx_v)
        r0 = wid * npw
        pltpu.async_copy(ys_hbm.at[pl.ds(r0 * CH, CH)], bufs[0], isems[0])
        if npw > 1:
            pltpu.async_copy(ys_hbm.at[pl.ds((r0 + 1) * CH, CH)], bufs[1],
                             isems[1])
        for c in range(npw):
            s = c % 2
            r = wid * npw + c
            pltpu.make_async_copy(
                ys_hbm.at[pl.ds(r * CH, CH)], bufs[s], isems[s]).wait()
            pltpu.async_copy(bufs[s], out_hbm.at[idx_v.at[c]], ssems[s])
            if c + 2 < npw:
                pltpu.make_async_copy(
                    bufs[s], out_hbm.at[idx_v.at[c]], ssems[s]).wait()
                pltpu.async_copy(ys_hbm.at[pl.ds((r + 2) * CH, CH)], bufs[s],
                                 isems[s])
        for c in range(max(npw - 2, 0), npw):
            s = c % 2
            pltpu.make_async_copy(
                bufs[s], out_hbm.at[idx_v.at[c]], ssems[s]).wait()

    return scatter


def kernel(x, Wg, bg, W1, b1, W2, b2, expert_bias):
    b, s, d = x.shape
    e_num, f_dim, _ = W1.shape
    t = b * s
    nt_data = t // TILE
    nt = nt_data + e_num
    tp = nt * TILE

    x_flat = x.reshape(t, d)

    wg_t = Wg.T
    bias2d = (bg + expert_bias).reshape(1, e_num)
    top = _gating(x_flat, wg_t, bias2d)

    i32 = jnp.int32
    perm = jnp.argsort(top).astype(i32)
    counts = jnp.bincount(top, length=e_num).astype(i32)
    offs = jnp.concatenate(
        [jnp.zeros((1,), i32), jnp.cumsum(counts)[:-1].astype(i32)])
    ptiles = (counts + TILE - 1) // TILE
    cumt = jnp.cumsum(ptiles).astype(i32)
    total_tiles = cumt[-1]
    tidx = jnp.arange(nt, dtype=i32)
    te = jnp.searchsorted(cumt, tidx, side="right").astype(i32)
    act = (tidx < total_tiles).astype(i32)
    e_last = jnp.take(te, total_tiles - 1)
    te = jnp.where(act == 1, te, e_last).astype(i32)
    poff = jnp.concatenate(
        [jnp.zeros((1,), i32),
         jnp.cumsum(ptiles * TILE)[:-1].astype(i32)])

    j = jnp.arange(tp, dtype=i32)
    tj = j // TILE
    ej = jnp.take(te, tj)
    lp = j - jnp.take(poff, ej)
    real = ((lp >= 0) & (lp < jnp.take(counts, ej))
            & (jnp.take(act, tj) == 1))
    src = jnp.take(perm, jnp.clip(jnp.take(offs, ej) + lp, 0, t - 1))
    gidx = jnp.where(real, src, j % t).astype(i32)
    sidx = jnp.where(real, src, t + (j % (tp - t))).astype(i32)

    npw = tp // (NW * CH)
    xs = _make_gather(t, d, tp, jnp.float32)(
        x_flat, gidx.reshape(NW, npw, CH)).astype(jnp.bfloat16)

    ys = _ffn(te, act, xs, W1, b1, W2, b2, nt)

    out_pad = _make_scatter(t, d, tp)(ys, sidx.reshape(NW, npw, CH))
    return out_pad[:t].reshape(b, s, d)
